# baseline (speedup 1.0000x reference)
import functools

import jax
import jax.numpy as jnp
from jax import lax
from jax.experimental import pallas as pl
from jax.experimental.pallas import tpu as pltpu
from jax.experimental.pallas import tpu_sc as plsc

_V = 1000
_D = 1000
_DP = 1024
_S = 1024
_T = 50
_NW = 32
_SPW = _S // _NW
_G = 16
_NGF = _T // _G
_REM = _T - _NGF * _G
_L = 16
_NVEC = 63


def _narrow_rows(src, dst, nrows):
    def row(r, carry):
        for m in range(_NVEC - 1):
            dst[r, pl.ds(m * _L, _L)] = src[r, pl.ds(m * _L, _L)]
        dst[r, pl.ds(_D - _L, _L)] = src[r, pl.ds(_D - _L, _L)]
        return carry

    lax.fori_loop(0, nrows, row, 0)


def _make_piece_kernel(nrows):
    def body(idx_hbm, table_hbm, out_hbm, idx_v, buf0, buf1, nar0, nar1,
             g0, g1, s0, s1):
        c = lax.axis_index("c")
        s = lax.axis_index("s")
        wid = s * 2 + c
        i0 = wid * _SPW
        pltpu.sync_copy(idx_hbm.at[pl.ds(i0, _SPW)], idx_v)

        bufs = (buf0, buf1)
        nars = (nar0, nar1)
        gsems = (g0, g1)
        ssems = (s0, s1)

        def gather(k, b):
            pltpu.async_copy(
                table_hbm.at[idx_v.at[k, pl.ds(0, nrows)]], bufs[b], gsems[b])

        def wait_gather(k, b):
            pltpu.make_async_copy(
                table_hbm.at[idx_v.at[k, pl.ds(0, nrows)]], bufs[b],
                gsems[b]).wait()

        def scatter(k, b):
            pltpu.async_copy(nars[b], out_hbm.at[i0 + k], ssems[b])

        def wait_scatter(k, b):
            pltpu.make_async_copy(nars[b], out_hbm.at[i0 + k],
                                  ssems[b]).wait()

        gather(0, 0)
        gather(1, 1)
        wait_gather(0, 0)
        _narrow_rows(bufs[0], nars[0], nrows)
        scatter(0, 0)
        wait_gather(1, 1)
        _narrow_rows(bufs[1], nars[1], nrows)
        scatter(1, 1)

        def loop(p, carry):
            for b in range(2):
                k = p * 2 + b
                wait_scatter(k - 2, b)
                gather(k, b)
                wait_gather(k, b)
                _narrow_rows(bufs[b], nars[b], nrows)
                scatter(k, b)
            return carry

        lax.fori_loop(1, _SPW // 2, loop, 0)
        wait_scatter(_SPW - 2, 0)
        wait_scatter(_SPW - 1, 1)

    mesh = plsc.VectorSubcoreMesh(core_axis_name="c", subcore_axis_name="s")
    return functools.partial(
        pl.kernel,
        mesh=mesh,
        out_type=jax.ShapeDtypeStruct((_S, nrows, _D), jnp.float32),
        scratch_types=[
            pltpu.VMEM((_SPW, _G), jnp.int32),
            pltpu.VMEM((nrows, _DP), jnp.float32),
            pltpu.VMEM((nrows, _DP), jnp.float32),
            pltpu.VMEM((nrows, _D), jnp.float32),
            pltpu.VMEM((nrows, _D), jnp.float32),
            pltpu.SemaphoreType.DMA,
            pltpu.SemaphoreType.DMA,
            pltpu.SemaphoreType.DMA,
            pltpu.SemaphoreType.DMA,
        ],
    )(body)


def _to6d(piece, nrows):
    p = piece.reshape(8, 128, nrows, 5, 25, 8)
    return p.transpose(2, 3, 4, 0, 5, 1)


@jax.jit
def _embedding_gather(seq, table_padded):
    full_piece = _make_piece_kernel(_G)
    rem_piece = _make_piece_kernel(_REM)
    pieces6d = []
    for jg in range(_NGF):
        idx = lax.slice_in_dim(seq, jg * _G, (jg + 1) * _G, axis=1)
        pieces6d.append(_to6d(full_piece(idx, table_padded), _G))
    idx_rem = jnp.concatenate(
        [seq[:, _NGF * _G:], jnp.tile(seq[:, -1:], (1, _G - _REM))], axis=1)
    pieces6d.append(_to6d(rem_piece(idx_rem, table_padded), _REM))
    out6d = jnp.concatenate(pieces6d, axis=0)
    return out6d.transpose(3, 5, 0, 1, 2, 4).reshape(_S, _T, _D)


def kernel(sequences, embedding):
    seq = sequences.astype(jnp.int32)
    table_padded = jnp.pad(embedding, ((0, 0), (0, _DP - _D)))
    return _embedding_gather(seq, table_padded)


# speedup vs baseline: 2.2384x; 2.2384x over previous
import functools

import jax
import jax.numpy as jnp
from jax import lax
from jax.experimental import pallas as pl
from jax.experimental.pallas import tpu as pltpu
from jax.experimental.pallas import tpu_sc as plsc

_V = 1000
_D = 1000
_DP = 1024
_S = 1024
_T = 50
_NW = 32
_SPW = _S // _NW
_G = 16
_NGF = _T // _G
_REM = _T - _NGF * _G
_GPS = _NGF + 1
_L = 16
_NVEC = 63
_NBUF = 3
_NSTEPS = _SPW * _GPS


def _narrow_rows(src, dst, nrows):
    def row(r, carry):
        for m in range(_NVEC - 1):
            dst[r, pl.ds(m * _L, _L)] = src[r, pl.ds(m * _L, _L)]
        dst[r, pl.ds(_D - _L, _L)] = src[r, pl.ds(_D - _L, _L)]
        return carry

    lax.fori_loop(0, nrows, row, 0)


def _gather_kernel(idx_hbm, table_hbm, out_hbm,
                   idx_v, b0, b1, b2, n0, n1, n2, rb0, rb1, rb2,
                   rn0, rn1, rn2, g0, g1, g2, s0, s1, s2):
    c = lax.axis_index("c")
    s = lax.axis_index("s")
    wid = s * 2 + c
    i0 = wid * _SPW
    pltpu.sync_copy(idx_hbm.at[pl.ds(wid * _SPW * _GPS, _SPW * _GPS)], idx_v)

    bufs = (b0, b1, b2)
    nars = (n0, n1, n2)
    rembufs = (rb0, rb1, rb2)
    remnars = (rn0, rn1, rn2)
    gsems = (g0, g1, g2)
    ssems = (s0, s1, s2)

    def _dispatch(k, full, remd):
        if isinstance(k, int):
            (full if k % _GPS < _NGF else remd)(0)
        else:
            lax.cond(k % _GPS < _NGF, full, remd, 0)

    def gather(k, b):
        slab = k // _GPS
        jg = k % _GPS

        def full(_):
            pltpu.async_copy(table_hbm.at[idx_v.at[slab * _GPS + jg]],
                             bufs[b], gsems[b])
            return 0

        def remd(_):
            pltpu.async_copy(
                table_hbm.at[idx_v.at[slab * _GPS + _NGF, pl.ds(0, _REM)]],
                rembufs[b], gsems[b])
            return 0

        _dispatch(k, full, remd)

    def wait_gather(k, b):
        slab = k // _GPS
        jg = k % _GPS

        def full(_):
            pltpu.make_async_copy(
                table_hbm.at[idx_v.at[slab * _GPS + jg]], bufs[b],
                gsems[b]).wait()
            return 0

        def remd(_):
            pltpu.make_async_copy(
                table_hbm.at[idx_v.at[slab * _GPS + _NGF, pl.ds(0, _REM)]],
                rembufs[b], gsems[b]).wait()
            return 0

        _dispatch(k, full, remd)

    def narrow(k, b):
        def full(_):
            _narrow_rows(bufs[b], nars[b], _G)
            return 0

        def remd(_):
            _narrow_rows(rembufs[b], remnars[b], _REM)
            return 0

        _dispatch(k, full, remd)

    def scatter(k, b):
        slab = k // _GPS
        jg = k % _GPS

        def full(_):
            pltpu.async_copy(
                nars[b], out_hbm.at[i0 + slab, pl.ds(jg * _G, _G), :],
                ssems[b])
            return 0

        def remd(_):
            pltpu.async_copy(
                remnars[b],
                out_hbm.at[i0 + slab, pl.ds(_NGF * _G, _REM), :], ssems[b])
            return 0

        _dispatch(k, full, remd)

    def wait_scatter(k, b):
        slab = k // _GPS
        jg = k % _GPS

        def full(_):
            pltpu.make_async_copy(
                nars[b], out_hbm.at[i0 + slab, pl.ds(jg * _G, _G), :],
                ssems[b]).wait()
            return 0

        def remd(_):
            pltpu.make_async_copy(
                remnars[b],
                out_hbm.at[i0 + slab, pl.ds(_NGF * _G, _REM), :],
                ssems[b]).wait()
            return 0

        _dispatch(k, full, remd)

    for b in range(_NBUF):
        gather(b, b)
    for k in range(_NBUF):
        b = k % _NBUF
        wait_gather(k, b)
        narrow(k, b)
        scatter(k, b)
        gather(k + _NBUF, b)

    def step_work(n, b):
        wait_scatter(n - _NBUF, b)
        wait_gather(n, b)
        narrow(n, b)
        scatter(n, b)
        gather(jnp.minimum(n + _NBUF, _NSTEPS - 1), b)
        return 0

    def step(n, carry):
        b = n % _NBUF
        lax.cond(b == 0,
                 lambda _: step_work(n, 0),
                 lambda _: lax.cond(b == 1,
                                    lambda __: step_work(n, 1),
                                    lambda __: step_work(n, 2), 0), 0)
        return carry

    lax.fori_loop(_NBUF, _NSTEPS, step, 0)

    for b in range(_NBUF):
        wait_gather(_NSTEPS - 1, b)
    for n in range(_NSTEPS - _NBUF, _NSTEPS):
        wait_scatter(n, n % _NBUF)


@jax.jit
def _embedding_gather(idx_groups, table2d):
    mesh = plsc.VectorSubcoreMesh(core_axis_name="c", subcore_axis_name="s")
    run = functools.partial(
        pl.kernel,
        mesh=mesh,
        out_type=jax.ShapeDtypeStruct((_S, _T, _D), jnp.float32),
        scratch_types=[
            pltpu.VMEM((_SPW * _GPS, _G), jnp.int32),
            pltpu.VMEM((_G, _DP), jnp.float32),
            pltpu.VMEM((_G, _DP), jnp.float32),
            pltpu.VMEM((_G, _DP), jnp.float32),
            pltpu.VMEM((_G, _D), jnp.float32),
            pltpu.VMEM((_G, _D), jnp.float32),
            pltpu.VMEM((_G, _D), jnp.float32),
            pltpu.VMEM((_REM, _DP), jnp.float32),
            pltpu.VMEM((_REM, _DP), jnp.float32),
            pltpu.VMEM((_REM, _DP), jnp.float32),
            pltpu.VMEM((_REM, _D), jnp.float32),
            pltpu.VMEM((_REM, _D), jnp.float32),
            pltpu.VMEM((_REM, _D), jnp.float32),
            pltpu.SemaphoreType.DMA,
            pltpu.SemaphoreType.DMA,
            pltpu.SemaphoreType.DMA,
            pltpu.SemaphoreType.DMA,
            pltpu.SemaphoreType.DMA,
            pltpu.SemaphoreType.DMA,
        ],
    )(_gather_kernel)
    return run(idx_groups, table2d)


def kernel(sequences, embedding):
    seq = sequences.astype(jnp.int32)
    full = seq[:, : _NGF * _G].reshape(_S, _NGF, _G)
    rem = jnp.concatenate(
        [seq[:, _NGF * _G :],
         jnp.tile(seq[:, -1:], (1, _G - _REM))], axis=1).reshape(_S, 1, _G)
    idx_groups = jnp.concatenate([full, rem], axis=1).reshape(_S * _GPS, _G)
    table2d = jnp.pad(embedding, ((0, 0), (0, _DP - _D)))
    return _embedding_gather(idx_groups, table2d)
